# baseline (speedup 1.0000x reference)
import jax
import jax.numpy as jnp
from jax import lax
from jax.experimental import pallas as pl
from jax.experimental.pallas import tpu as pltpu
from jax.experimental.pallas import tpu_sc as plsc

_B, _C, _L = 64, 64, 16384
_R = _B * _C
_NW = 32
_GPW = 16
_NK = 8
_CW = _L // _NK
_NT = _CW // 128
_SW = _CW + 256
_SRC = [min(max(k * _CW - 128, 0), _L - _SW) for k in range(_NK)]


def _body(x_hbm, ph_hbm, out_hbm, in0, in1, ot0, ot1, phv,
          isem0, isem1, osem0, osem1):
  ncores = 2
  wid = lax.axis_index("s") * ncores + lax.axis_index("c")
  base_row = wid * _GPW * 8

  pltpu.sync_copy(ph_hbm.at[pl.ds(base_row, _GPW * 8)], phv)

  iota = lax.broadcasted_iota(jnp.int32, (16,), 0)
  rrvs = [jnp.broadcast_to(rr, (16,)) for rr in range(8)]
  ins = (in0, in1)
  outs = (ot0, ot1)
  isems = (isem0, isem1)
  osems = (osem0, osem1)

  def in_start(gi, k, s):
    row0 = base_row + gi * 8
    pltpu.async_copy(x_hbm.at[pl.ds(row0, 8), pl.ds(_SRC[k], _SW)],
                     ins[s], isems[s])

  def in_wait(s):
    pltpu.make_async_copy(x_hbm.at[pl.ds(0, 8), pl.ds(0, _SW)],
                          ins[s], isems[s]).wait()

  def out_start(gi, k, s):
    row0 = base_row + gi * 8
    pltpu.async_copy(outs[s], out_hbm.at[pl.ds(row0, 8),
                                         pl.ds(k * _CW, _CW)], osems[s])

  def out_wait(s):
    pltpu.make_async_copy(outs[s], out_hbm.at[pl.ds(0, 8), pl.ds(0, _CW)],
                          osems[s]).wait()

  in_start(0, 0, 0)
  in_start(0, 1, 1)

  @pl.loop(0, _GPW)
  def _group(gi):
    pvs = [plsc.load_gather(phv, [jnp.broadcast_to(gi * 8 + rr, (16,))])
           for rr in range(8)]
    vms = [pv + iota for pv in pvs]

    for k in range(_NK):
      s = k % 2
      it = gi * _NK + k
      in_wait(s)

      @pl.when(it >= 2)
      def _():
        out_wait(s)

      inb, outb = ins[s], outs[s]
      w0 = k * _CW - _SRC[k]

      def tile_body(j, k=k, inb=inb, outb=outb):
        src = inb.at[:, pl.ds(pl.multiple_of(j * 128 + w0, 128), 128)]
        dst = outb.at[:, pl.ds(pl.multiple_of(j * 128, 128), 128)]
        us = range(1, 8) if isinstance(j, int) and k == 0 and j == 0 else (
            range(7) if isinstance(j, int) and k == _NK - 1 and j == _NT - 1
            else range(8))
        for rr in range(8):
          for u in us:
            vals = plsc.load_gather(src, [rrvs[rr], vms[rr] + u * 16])
            plsc.store_scatter(dst, [rrvs[rr], iota + u * 16], vals)

      if k == 0:
        tile_body(0)
        pl.loop(1, _NT)(tile_body)
        for rr in range(8):
          q = vms[rr]
          qr = jnp.where(q < 0, -q, q)
          plsc.store_scatter(outb, [rrvs[rr], iota],
                             plsc.load_gather(inb, [rrvs[rr], qr]))
      elif k == _NK - 1:
        pl.loop(0, _NT - 1)(tile_body)
        tile_body(_NT - 1)
        for rr in range(8):
          q = vms[rr] + (_L - 16)
          qr = jnp.where(q >= _L, 2 * (_L - 1) - q, q)
          plsc.store_scatter(outb, [rrvs[rr], iota + (_CW - 16)],
                             plsc.load_gather(inb, [rrvs[rr], qr - _SRC[k]]))
      else:
        pl.loop(0, _NT)(tile_body)

      out_start(gi, k, s)

      @pl.when(it + 2 < _GPW * _NK)
      def _():
        gi_next = gi + 1 if k >= _NK - 2 else gi
        in_start(gi_next, (k + 2) % _NK, s)

  out_wait(0)
  out_wait(1)


@jax.jit
def kernel(x, phase_offsets):
  xr = x.reshape(_R, _L)
  ph = phase_offsets.reshape(_R).astype(jnp.int32)

  mesh = plsc.VectorSubcoreMesh(core_axis_name="c", subcore_axis_name="s")
  run = pl.kernel(
      _body,
      out_type=jax.ShapeDtypeStruct((_R, _L), jnp.float32),
      mesh=mesh,
      compiler_params=pltpu.CompilerParams(needs_layout_passes=False,
                                           use_tc_tiling_on_sc=True),
      scratch_types=[
          pltpu.VMEM((8, _SW), jnp.float32),
          pltpu.VMEM((8, _SW), jnp.float32),
          pltpu.VMEM((8, _CW), jnp.float32),
          pltpu.VMEM((8, _CW), jnp.float32),
          pltpu.VMEM((_GPW * 8,), jnp.int32),
          pltpu.SemaphoreType.DMA,
          pltpu.SemaphoreType.DMA,
          pltpu.SemaphoreType.DMA,
          pltpu.SemaphoreType.DMA,
      ],
  )
  out = run(xr, ph)
  return out.reshape(_B, _C, _L)


# speedup vs baseline: 3.4691x; 3.4691x over previous
import jax
import jax.numpy as jnp
from jax import lax
from jax.experimental import pallas as pl
from jax.experimental.pallas import tpu as pltpu
from jax.experimental.pallas import tpu_sc as plsc

_B, _C, _L = 64, 64, 16384
_R = _B * _C
_NW = 32
_GPW = 16
_NK = 8
_CW = _L // _NK
_NT = _CW // 128
_SW = _CW + 256
_SRC = [min(max(k * _CW - 128, 0), _L - _SW) for k in range(_NK)]


def _body(x_hbm, ph_hbm, out_hbm, in0, in1, ot0, ot1, phv,
          isem0, isem1, osem0, osem1):
  ncores = 2
  wid = lax.axis_index("s") * ncores + lax.axis_index("c")
  base_row = wid * _GPW * 8

  pltpu.sync_copy(ph_hbm.at[pl.ds(base_row, _GPW * 8)], phv)

  iota = lax.broadcasted_iota(jnp.int32, (16,), 0)
  rrvs = [jnp.broadcast_to(rr, (16,)) for rr in range(8)]
  ins = (in0, in1)
  outs = (ot0, ot1)
  isems = (isem0, isem1)
  osems = (osem0, osem1)

  def in_start(gi, k, s):
    row0 = base_row + gi * 8
    pltpu.async_copy(x_hbm.at[pl.ds(row0, 8), pl.ds(_SRC[k], _SW)],
                     ins[s], isems[s])

  def in_wait(s):
    pltpu.make_async_copy(x_hbm.at[pl.ds(0, 8), pl.ds(0, _SW)],
                          ins[s], isems[s]).wait()

  def out_start(gi, k, s):
    row0 = base_row + gi * 8
    pltpu.async_copy(outs[s], out_hbm.at[pl.ds(row0, 8),
                                         pl.ds(k * _CW, _CW)], osems[s])

  def out_wait(s):
    pltpu.make_async_copy(outs[s], out_hbm.at[pl.ds(0, 8), pl.ds(0, _CW)],
                          osems[s]).wait()

  in_start(0, 0, 0)
  in_start(0, 1, 1)

  @pl.loop(0, _GPW)
  def _group(gi):
    pvs = [plsc.load_gather(phv, [jnp.broadcast_to(gi * 8 + rr, (16,))])
           for rr in range(8)]
    vms = [pv + iota for pv in pvs]

    for k in range(_NK):
      s = k % 2
      it = gi * _NK + k
      in_wait(s)

      @pl.when(it >= 2)
      def _():
        out_wait(s)

      inb, outb = ins[s], outs[s]
      w0 = k * _CW - _SRC[k]

      for rr in range(8):
        rrv = rrvs[rr]
        lidx = [vms[rr] + u * 16 for u in range(8)]

        def tile_us(j, k=k):
          if isinstance(j, int) and k == 0 and j == 0:
            return range(1, 8)
          if isinstance(j, int) and k == _NK - 1 and j == _NT - 1:
            return range(7)
          return range(8)

        def _src(j, inb=inb):
          return inb.at[:, pl.ds(pl.multiple_of(j * 128 + w0, 128), 128)]

        def _dst(j, outb=outb):
          return outb.at[:, pl.ds(pl.multiple_of(j * 128, 128), 128)]

        def tile_body(*js, rrv=rrv, lidx=lidx):
          vals = []
          for j in js:
            vals.append([plsc.load_gather(_src(j), [rrv, lidx[u]])
                         for u in tile_us(j)])
          for j, vj in zip(js, vals):
            for u, v in zip(tile_us(j), vj):
              plsc.store_scatter(_dst(j), [rrv, iota + u * 16], v)

        def tile_pair(j2, rrv=rrv, lidx=lidx):
          a, b = j2 * 2, j2 * 2 + 1
          va = [plsc.load_gather(_src(a), [rrv, lidx[u]]) for u in range(8)]
          vb = []
          for u in range(8):
            plsc.store_scatter(_dst(a), [rrv, iota + u * 16], va[u])
            vb.append(plsc.load_gather(_src(b), [rrv, lidx[u]]))
          for u in range(8):
            plsc.store_scatter(_dst(b), [rrv, iota + u * 16], vb[u])

        if k == 0:
          tile_body(0, 1)
          pl.loop(1, _NT // 2)(tile_pair)
          q = vms[rr]
          qr = jnp.where(q < 0, -q, q)
          plsc.store_scatter(outb, [rrv, iota],
                             plsc.load_gather(inb, [rrv, qr]))
        elif k == _NK - 1:
          pl.loop(0, _NT // 2 - 1)(tile_pair)
          tile_body(_NT - 2, _NT - 1)
          q = vms[rr] + (_L - 16)
          qr = jnp.where(q >= _L, 2 * (_L - 1) - q, q)
          plsc.store_scatter(outb, [rrv, iota + (_CW - 16)],
                             plsc.load_gather(inb, [rrv, qr - _SRC[k]]))
        else:
          pl.loop(0, _NT // 2)(tile_pair)

      out_start(gi, k, s)

      @pl.when(it + 2 < _GPW * _NK)
      def _():
        gi_next = gi + 1 if k >= _NK - 2 else gi
        in_start(gi_next, (k + 2) % _NK, s)

  out_wait(0)
  out_wait(1)


@jax.jit
def kernel(x, phase_offsets):
  xr = x.reshape(_R, _L)
  ph = phase_offsets.reshape(_R).astype(jnp.int32)

  mesh = plsc.VectorSubcoreMesh(core_axis_name="c", subcore_axis_name="s")
  run = pl.kernel(
      _body,
      out_type=jax.ShapeDtypeStruct((_R, _L), jnp.float32),
      mesh=mesh,
      compiler_params=pltpu.CompilerParams(needs_layout_passes=False,
                                           use_tc_tiling_on_sc=True),
      scratch_types=[
          pltpu.VMEM((8, _SW), jnp.float32),
          pltpu.VMEM((8, _SW), jnp.float32),
          pltpu.VMEM((8, _CW), jnp.float32),
          pltpu.VMEM((8, _CW), jnp.float32),
          pltpu.VMEM((_GPW * 8,), jnp.int32),
          pltpu.SemaphoreType.DMA,
          pltpu.SemaphoreType.DMA,
          pltpu.SemaphoreType.DMA,
          pltpu.SemaphoreType.DMA,
      ],
  )
  out = run(xr, ph)
  return out.reshape(_B, _C, _L)


# speedup vs baseline: 3.6526x; 1.0529x over previous
import jax
import jax.numpy as jnp
from jax import lax
from jax.experimental import pallas as pl
from jax.experimental.pallas import tpu as pltpu
from jax.experimental.pallas import tpu_sc as plsc

_B, _C, _L = 64, 64, 16384
_R = _B * _C
_NW = 32
_GPW = 16
_NK = 8
_CW = _L // _NK
_NT = _CW // 128
_SW = _CW + 256
_SRC = [min(max(k * _CW - 128, 0), _L - _SW) for k in range(_NK)]


def _body(x_hbm, ph_hbm, out_hbm, in0, in1, in2, in3, ot0, ot1, phv,
          isem0, isem1, isem2, isem3, osem0, osem1):
  ncores = 2
  wid = lax.axis_index("s") * ncores + lax.axis_index("c")
  base_row = wid * _GPW * 8

  pltpu.sync_copy(ph_hbm.at[pl.ds(base_row, _GPW * 8)], phv)

  iota = lax.broadcasted_iota(jnp.int32, (16,), 0)
  rrvs = [jnp.broadcast_to(rr, (16,)) for rr in range(8)]
  ins = (in0, in1, in2, in3)
  outs = (ot0, ot1)
  isems = (isem0, isem1, isem2, isem3)
  osems = (osem0, osem1)

  def in_start(gi, k, s):
    row0 = base_row + gi * 8
    pltpu.async_copy(x_hbm.at[pl.ds(row0, 8), pl.ds(_SRC[k], _SW)],
                     ins[s], isems[s])

  def in_wait(s):
    pltpu.make_async_copy(x_hbm.at[pl.ds(0, 8), pl.ds(0, _SW)],
                          ins[s], isems[s]).wait()

  def out_start(gi, k, s):
    row0 = base_row + gi * 8
    pltpu.async_copy(outs[s], out_hbm.at[pl.ds(row0, 8),
                                         pl.ds(k * _CW, _CW)], osems[s])

  def out_wait(s):
    pltpu.make_async_copy(outs[s], out_hbm.at[pl.ds(0, 8), pl.ds(0, _CW)],
                          osems[s]).wait()

  in_start(0, 0, 0)
  in_start(0, 1, 1)
  in_start(0, 2, 2)
  in_start(0, 3, 3)

  @pl.loop(0, _GPW)
  def _group(gi):
    pvs = [plsc.load_gather(phv, [jnp.broadcast_to(gi * 8 + rr, (16,))])
           for rr in range(8)]
    vms = [pv + iota for pv in pvs]

    for k in range(_NK):
      si = k % 4
      s = k % 2
      it = gi * _NK + k
      in_wait(si)

      @pl.when(it >= 2)
      def _():
        out_wait(s)

      inb, outb = ins[si], outs[s]
      w0 = k * _CW - _SRC[k]

      for rr in range(8):
        rrv = rrvs[rr]
        lidx = [vms[rr] + u * 16 for u in range(8)]

        def tile_us(j, k=k):
          if isinstance(j, int) and k == 0 and j == 0:
            return range(1, 8)
          if isinstance(j, int) and k == _NK - 1 and j == _NT - 1:
            return range(7)
          return range(8)

        def _src(j, inb=inb):
          return inb.at[:, pl.ds(pl.multiple_of(j * 128 + w0, 128), 128)]

        def _dst(j, outb=outb):
          return outb.at[:, pl.ds(pl.multiple_of(j * 128, 128), 128)]

        def tile_body(*js, rrv=rrv, lidx=lidx):
          vals = []
          for j in js:
            vals.append([plsc.load_gather(_src(j), [rrv, lidx[u]])
                         for u in tile_us(j)])
          for j, vj in zip(js, vals):
            for u, v in zip(tile_us(j), vj):
              plsc.store_scatter(_dst(j), [rrv, iota + u * 16], v)

        def tile_pair(j2, rrv=rrv, lidx=lidx):
          a, b = j2 * 2, j2 * 2 + 1
          va = [plsc.load_gather(_src(a), [rrv, lidx[u]]) for u in range(8)]
          vb = []
          for u in range(8):
            plsc.store_scatter(_dst(a), [rrv, iota + u * 16], va[u])
            vb.append(plsc.load_gather(_src(b), [rrv, lidx[u]]))
          for u in range(8):
            plsc.store_scatter(_dst(b), [rrv, iota + u * 16], vb[u])

        if k == 0:
          tile_body(0, 1)
          pl.loop(1, _NT // 2)(tile_pair)
          q = vms[rr]
          qr = jnp.where(q < 0, -q, q)
          plsc.store_scatter(outb, [rrv, iota],
                             plsc.load_gather(inb, [rrv, qr]))
        elif k == _NK - 1:
          pl.loop(0, _NT // 2 - 1)(tile_pair)
          tile_body(_NT - 2, _NT - 1)
          q = vms[rr] + (_L - 16)
          qr = jnp.where(q >= _L, 2 * (_L - 1) - q, q)
          plsc.store_scatter(outb, [rrv, iota + (_CW - 16)],
                             plsc.load_gather(inb, [rrv, qr - _SRC[k]]))
        else:
          pl.loop(0, _NT // 2)(tile_pair)

      out_start(gi, k, s)

      @pl.when(it + 4 < _GPW * _NK)
      def _():
        gi_next = gi + 1 if k >= _NK - 4 else gi
        in_start(gi_next, (k + 4) % _NK, si)

  out_wait(0)
  out_wait(1)


@jax.jit
def kernel(x, phase_offsets):
  xr = x.reshape(_R, _L)
  ph = phase_offsets.reshape(_R).astype(jnp.int32)

  mesh = plsc.VectorSubcoreMesh(core_axis_name="c", subcore_axis_name="s")
  run = pl.kernel(
      _body,
      out_type=jax.ShapeDtypeStruct((_R, _L), jnp.float32),
      mesh=mesh,
      compiler_params=pltpu.CompilerParams(needs_layout_passes=False,
                                           use_tc_tiling_on_sc=True),
      scratch_types=[
          pltpu.VMEM((8, _SW), jnp.float32),
          pltpu.VMEM((8, _SW), jnp.float32),
          pltpu.VMEM((8, _SW), jnp.float32),
          pltpu.VMEM((8, _SW), jnp.float32),
          pltpu.VMEM((8, _CW), jnp.float32),
          pltpu.VMEM((8, _CW), jnp.float32),
          pltpu.VMEM((_GPW * 8,), jnp.int32),
          pltpu.SemaphoreType.DMA,
          pltpu.SemaphoreType.DMA,
          pltpu.SemaphoreType.DMA,
          pltpu.SemaphoreType.DMA,
          pltpu.SemaphoreType.DMA,
          pltpu.SemaphoreType.DMA,
      ],
  )
  out = run(xr, ph)
  return out.reshape(_B, _C, _L)
